# baseline (speedup 1.0000x reference)
import jax
import jax.numpy as jnp
from jax.experimental import pallas as pl

_N = 8
_I = 128
_TB = 512


def _moe_body(x_ref, wr_ref, wg_ref, bg_ref, wu_ref, eb_ref, o_ref):
    xb = x_ref[...]
    tb = xb.shape[0]

    logits_t = jax.lax.dot_general(
        wr_ref[...], xb, (((1,), (1,)), ((), ())),
        preferred_element_type=jnp.float32)
    n_iota = jax.lax.broadcasted_iota(jnp.int32, (_N, tb), 0)
    m1 = jnp.max(logits_t, axis=0, keepdims=True)
    i1 = jnp.min(jnp.where(logits_t == m1, n_iota, _N), axis=0, keepdims=True)
    masked = jnp.where(n_iota == i1, -jnp.inf, logits_t)
    m2 = jnp.max(masked, axis=0, keepdims=True)
    i2 = jnp.min(jnp.where(masked == m2, n_iota, _N), axis=0, keepdims=True)
    w1 = jax.nn.sigmoid(m1 - m2)
    comb_t = (jnp.where(n_iota == i1, w1, 0.0)
              + jnp.where(n_iota == i2, 1.0 - w1, 0.0))

    eb = jax.lax.dot_general(comb_t, eb_ref[...], (((0,), (0,)), ((), ())),
                             preferred_element_type=jnp.float32)
    cw = eb[:, :_N * _I]
    bu_term = eb[:, _N * _I:]

    h = jax.lax.dot_general(xb.astype(jnp.bfloat16), wg_ref[...],
                            (((1,), (1,)), ((), ())),
                            preferred_element_type=jnp.float32)
    h = h + bg_ref[...]
    h = h * jax.nn.sigmoid(h)
    h = (h * cw).astype(jnp.bfloat16)

    acc = bu_term
    for n in range(_N):
        acc = acc + jax.lax.dot_general(
            h[:, n * _I:(n + 1) * _I], wu_ref[n],
            (((1,), (1,)), ((), ())),
            preferred_element_type=jnp.float32)
    o_ref[...] = acc


def kernel(x, Wr, Wg, bg, Wu, bu):
    b, s, h = x.shape
    t = b * s
    xf = x.reshape(t, h)
    wg2 = Wg.reshape(_N * _I, h).astype(jnp.bfloat16)
    wub = Wu.astype(jnp.bfloat16)
    bg1 = bg.reshape(1, _N * _I)
    expand = (jnp.arange(_N * _I, dtype=jnp.int32)[None, :] // _I
              == jnp.arange(_N, dtype=jnp.int32)[:, None]).astype(jnp.float32)
    eb = jnp.concatenate([expand, bu], axis=1)

    out = pl.pallas_call(
        _moe_body,
        grid=(t // _TB,),
        in_specs=[
            pl.BlockSpec((_TB, h), lambda i: (i, 0)),
            pl.BlockSpec((_N, h), lambda i: (0, 0)),
            pl.BlockSpec((_N * _I, h), lambda i: (0, 0)),
            pl.BlockSpec((1, _N * _I), lambda i: (0, 0)),
            pl.BlockSpec((_N, h, _I), lambda i: (0, 0, 0)),
            pl.BlockSpec((_N, _N * _I + h), lambda i: (0, 0)),
        ],
        out_specs=pl.BlockSpec((_TB, h), lambda i: (i, 0)),
        out_shape=jax.ShapeDtypeStruct((t, h), jnp.float32),
    )(xf, Wr, wg2, bg1, wub, eb)
    return out.reshape(b, s, h)


# speedup vs baseline: 1.1088x; 1.1088x over previous
import jax
import jax.numpy as jnp
from jax.experimental import pallas as pl
from jax.experimental.pallas import tpu as pltpu

_N = 8
_I = 128
_TB = 512


def _moe_body(x_ref, wr_ref, wg_ref, bg_ref, wu_ref, eb_ref, o_ref,
              wgb_ref, wub_ref):
    xb = x_ref[...]
    tb = xb.shape[0]

    @pl.when(pl.program_id(0) == 0)
    def _cast_weights():
        wgb_ref[...] = wg_ref[...].astype(jnp.bfloat16)
        wub_ref[...] = wu_ref[...].astype(jnp.bfloat16)

    logits_t = jax.lax.dot_general(
        wr_ref[...], xb, (((1,), (1,)), ((), ())),
        preferred_element_type=jnp.float32)
    n_iota = jax.lax.broadcasted_iota(jnp.int32, (_N, tb), 0)
    m1 = jnp.max(logits_t, axis=0, keepdims=True)
    i1 = jnp.min(jnp.where(logits_t == m1, n_iota, _N), axis=0, keepdims=True)
    masked = jnp.where(n_iota == i1, -jnp.inf, logits_t)
    m2 = jnp.max(masked, axis=0, keepdims=True)
    i2 = jnp.min(jnp.where(masked == m2, n_iota, _N), axis=0, keepdims=True)
    w1 = jax.nn.sigmoid(m1 - m2)
    comb_t = (jnp.where(n_iota == i1, w1, 0.0)
              + jnp.where(n_iota == i2, 1.0 - w1, 0.0))

    eb = jax.lax.dot_general(comb_t, eb_ref[...], (((0,), (0,)), ((), ())),
                             preferred_element_type=jnp.float32)
    cw = eb[:, :_N * _I]
    bu_term = eb[:, _N * _I:]

    h = jax.lax.dot_general(xb.astype(jnp.bfloat16), wgb_ref[...],
                            (((1,), (1,)), ((), ())),
                            preferred_element_type=jnp.float32)
    h = h + bg_ref[...]
    h = h * jax.nn.sigmoid(h)
    h = (h * cw).astype(jnp.bfloat16)

    acc = bu_term
    for n in range(_N):
        acc = acc + jax.lax.dot_general(
            h[:, n * _I:(n + 1) * _I], wub_ref[n],
            (((1,), (1,)), ((), ())),
            preferred_element_type=jnp.float32)
    o_ref[...] = acc


def kernel(x, Wr, Wg, bg, Wu, bu):
    b, s, h = x.shape
    t = b * s
    xf = x.reshape(t, h)
    wg2 = Wg.reshape(_N * _I, h)
    bg1 = bg.reshape(1, _N * _I)
    expand = (jnp.arange(_N * _I, dtype=jnp.int32)[None, :] // _I
              == jnp.arange(_N, dtype=jnp.int32)[:, None]).astype(jnp.float32)
    eb = jnp.concatenate([expand, bu], axis=1)

    out = pl.pallas_call(
        _moe_body,
        grid=(t // _TB,),
        in_specs=[
            pl.BlockSpec((_TB, h), lambda i: (i, 0)),
            pl.BlockSpec((_N, h), lambda i: (0, 0)),
            pl.BlockSpec((_N * _I, h), lambda i: (0, 0)),
            pl.BlockSpec((1, _N * _I), lambda i: (0, 0)),
            pl.BlockSpec((_N, h, _I), lambda i: (0, 0, 0)),
            pl.BlockSpec((_N, _N * _I + h), lambda i: (0, 0)),
        ],
        out_specs=pl.BlockSpec((_TB, h), lambda i: (i, 0)),
        out_shape=jax.ShapeDtypeStruct((t, h), jnp.float32),
        scratch_shapes=[
            pltpu.VMEM((_N * _I, h), jnp.bfloat16),
            pltpu.VMEM((_N, h, _I), jnp.bfloat16),
        ],
    )(xf, Wr, wg2, bg1, Wu, eb)
    return out.reshape(b, s, h)


# speedup vs baseline: 1.3421x; 1.2103x over previous
import jax
import jax.numpy as jnp
from jax.experimental import pallas as pl
from jax.experimental.pallas import tpu as pltpu

_N = 8
_I = 128
_TB = 512


def _moe_body(x_ref, wr_ref, wg_ref, bg_ref, wu_ref, eb_ref, o_ref,
              wgb_ref, wub_ref):
    xb = x_ref[...]
    tb = xb.shape[0]

    @pl.when(pl.program_id(0) == 0)
    def _cast_weights():
        wgb_ref[...] = wg_ref[...].astype(jnp.bfloat16)
        for p in range(_N // 2):
            wub_ref[p, :, :_I] = wu_ref[2 * p].astype(jnp.bfloat16)
            wub_ref[p, :, _I:] = wu_ref[2 * p + 1].astype(jnp.bfloat16)

    logits_t = jax.lax.dot_general(
        wr_ref[...], xb, (((1,), (1,)), ((), ())),
        preferred_element_type=jnp.float32)
    n_iota = jax.lax.broadcasted_iota(jnp.int32, (_N, tb), 0)
    m1 = jnp.max(logits_t, axis=0, keepdims=True)
    i1 = jnp.min(jnp.where(logits_t == m1, n_iota, _N), axis=0, keepdims=True)
    masked = jnp.where(n_iota == i1, -jnp.inf, logits_t)
    m2 = jnp.max(masked, axis=0, keepdims=True)
    i2 = jnp.min(jnp.where(masked == m2, n_iota, _N), axis=0, keepdims=True)
    w1 = jax.nn.sigmoid(m1 - m2)
    comb_t = (jnp.where(n_iota == i1, w1, 0.0)
              + jnp.where(n_iota == i2, 1.0 - w1, 0.0))

    eb = jax.lax.dot_general(comb_t, eb_ref[...], (((0,), (0,)), ((), ())),
                             preferred_element_type=jnp.float32)
    cw = eb[:, :_N * _I]
    bu_term = eb[:, _N * _I:]

    h = jax.lax.dot_general(xb.astype(jnp.bfloat16), wgb_ref[...],
                            (((1,), (1,)), ((), ())),
                            preferred_element_type=jnp.float32)
    h = h + bg_ref[...]
    h = h * jax.nn.sigmoid(h)
    h = (h * cw).astype(jnp.bfloat16)

    acc = bu_term
    for p in range(_N // 2):
        acc = acc + jax.lax.dot_general(
            h[:, 2 * p * _I:2 * (p + 1) * _I], wub_ref[p],
            (((1,), (1,)), ((), ())),
            preferred_element_type=jnp.float32)
    o_ref[...] = acc


def kernel(x, Wr, Wg, bg, Wu, bu):
    b, s, h = x.shape
    t = b * s
    xf = x.reshape(t, h)
    wg2 = Wg.reshape(_N * _I, h)
    bg1 = bg.reshape(1, _N * _I)
    expand = (jnp.arange(_N * _I, dtype=jnp.int32)[None, :] // _I
              == jnp.arange(_N, dtype=jnp.int32)[:, None]).astype(jnp.float32)
    eb = jnp.concatenate([expand, bu], axis=1)

    out = pl.pallas_call(
        _moe_body,
        grid=(t // _TB,),
        in_specs=[
            pl.BlockSpec((_TB, h), lambda i: (i, 0)),
            pl.BlockSpec((_N, h), lambda i: (0, 0)),
            pl.BlockSpec((_N * _I, h), lambda i: (0, 0)),
            pl.BlockSpec((1, _N * _I), lambda i: (0, 0)),
            pl.BlockSpec((_N, h, _I), lambda i: (0, 0, 0)),
            pl.BlockSpec((_N, _N * _I + h), lambda i: (0, 0)),
        ],
        out_specs=pl.BlockSpec((_TB, h), lambda i: (i, 0)),
        out_shape=jax.ShapeDtypeStruct((t, h), jnp.float32),
        scratch_shapes=[
            pltpu.VMEM((_N * _I, h), jnp.bfloat16),
            pltpu.VMEM((_N // 2, h, 2 * _I), jnp.bfloat16),
        ],
    )(xf, Wr, wg2, bg1, Wu, eb)
    return out.reshape(b, s, h)
